# baseline (device time: 1211547 ns/iter reference)
import jax
import jax.numpy as jnp
from jax import lax
from jax.experimental import pallas as pl
from jax.experimental.pallas import tpu as pltpu

N_DEV = 4
M = 4096
K = 1024
N = 8192
HALF = N // 2
CHUNK = M // N_DEV
PIECE = 512
PIECES = CHUNK // PIECE
N_STEP = N_DEV - 1


def _gelu(y):
    c = 0.7978845608028654
    return 0.5 * y * (1.0 + jnp.tanh(c * (y + 0.044715 * y * y * y)))


def kernel(x, w_mat):
    me_out = lax.axis_index("i")
    xs = lax.dynamic_slice_in_dim(x, me_out * CHUNK, CHUNK, axis=0)
    p0 = jnp.dot(xs, w_mat, preferred_element_type=jnp.float32)

    def body(x_ref, w_ref, p0_ref, out_ref, sb0, sb1, rb0, rb1, va, vx,
             rs_send, rs_recv, ag_send, ag_recv, sem_a, sem_x, sem_o):
        me = lax.axis_index("i")
        right = lax.rem(me + 1, N_DEV)
        left = lax.rem(me + N_DEV - 1, N_DEV)
        ring_nbr = (right, left)
        ring_col0 = (0, HALF)
        ring_rb = (rb0, rb1)
        ring_sb = (sb0, sb1)

        barrier = pltpu.get_barrier_semaphore()
        for nbr in (left, right):
            pl.semaphore_signal(barrier, inc=1, device_id=(nbr,),
                                device_id_type=pl.DeviceIdType.MESH)
        pl.semaphore_wait(barrier, 2)

        def accum_piece(src_piece, row0, col0, dest_piece, apply_gelu):
            ca = pltpu.make_async_copy(src_piece, va, sem_a)
            cx = pltpu.make_async_copy(
                x_ref.at[pl.ds(row0, PIECE), :], vx, sem_x)
            ca.start()
            cx.start()
            cx.wait()
            pp = jnp.dot(vx[...], w_ref[:, pl.ds(col0, HALF)],
                         preferred_element_type=jnp.float32)
            ca.wait()
            s = va[...] + pp
            va[...] = _gelu(s) if apply_gelu else s
            co = pltpu.make_async_copy(va, dest_piece, sem_o)
            co.start()
            co.wait()

        descs = {}
        ag_descs = {}

        def start_send(r, s, j, src_piece):
            d = pltpu.make_async_remote_copy(
                src_ref=src_piece,
                dst_ref=ring_rb[r].at[s, pl.ds(j * PIECE, PIECE), :],
                send_sem=rs_send.at[r, s, j],
                recv_sem=rs_recv.at[r, s, j],
                device_id=(ring_nbr[r],),
                device_id_type=pl.DeviceIdType.MESH,
            )
            d.start()
            descs[(r, s, j)] = d

        def start_ag(r, t, j):
            if r == 0:
                g = lax.rem(me + 1 - t + N_DEV, N_DEV)
            else:
                g = lax.rem(me - 1 + t + N_DEV, N_DEV)
            piece = out_ref.at[pl.ds(g * CHUNK + j * PIECE, PIECE),
                               pl.ds(ring_col0[r], HALF)]
            d = pltpu.make_async_remote_copy(
                src_ref=piece, dst_ref=piece,
                send_sem=ag_send.at[r, t, j],
                recv_sem=ag_recv.at[r, t, j],
                device_id=(ring_nbr[r],),
                device_id_type=pl.DeviceIdType.MESH,
            )
            d.start()
            ag_descs[(r, t, j)] = d

        for j in range(PIECES):
            for r in (0, 1):
                start_send(r, 0, j, p0_ref.at[pl.ds(j * PIECE, PIECE),
                                              pl.ds(ring_col0[r], HALF)])

        for s in range(N_STEP):
            last = s == N_STEP - 1
            rc = (lax.rem(me - s - 1 + N_DEV, N_DEV),
                  lax.rem(me + s + 1, N_DEV))
            for j in range(PIECES):
                for r in (0, 1):
                    descs[(r, s, j)].wait_recv()
                    src = ring_rb[r].at[s, pl.ds(j * PIECE, PIECE), :]
                    row0 = rc[r] * CHUNK + j * PIECE
                    if not last:
                        dest = ring_sb[r].at[s, pl.ds(j * PIECE, PIECE), :]
                        accum_piece(src, row0, ring_col0[r], dest, False)
                        start_send(r, s + 1, j, dest)
                    else:
                        dest = out_ref.at[pl.ds(row0, PIECE),
                                          pl.ds(ring_col0[r], HALF)]
                        accum_piece(src, row0, ring_col0[r], dest, True)
                        start_ag(r, 0, j)
        for d in descs.values():
            d.wait_send()

        for t in range(1, N_STEP):
            for j in range(PIECES):
                for r in (0, 1):
                    ag_descs[(r, t - 1, j)].wait_recv()
                    start_ag(r, t, j)
        for j in range(PIECES):
            for r in (0, 1):
                ag_descs[(r, N_STEP - 1, j)].wait_recv()
        for d in ag_descs.values():
            d.wait_send()

        def _exit(second_barrier):
            for nbr in (left, right):
                pl.semaphore_signal(second_barrier, inc=1, device_id=(nbr,),
                                    device_id_type=pl.DeviceIdType.MESH)
            pl.semaphore_wait(second_barrier, 2)
        pl.run_scoped(_exit, second_barrier=pltpu.SemaphoreType.REGULAR)

    out = pl.pallas_call(
        body,
        out_shape=[
            jax.ShapeDtypeStruct((M, N), jnp.float32),
            jax.ShapeDtypeStruct((N_STEP - 1, CHUNK, HALF), jnp.float32),
            jax.ShapeDtypeStruct((N_STEP - 1, CHUNK, HALF), jnp.float32),
            jax.ShapeDtypeStruct((N_STEP, CHUNK, HALF), jnp.float32),
            jax.ShapeDtypeStruct((N_STEP, CHUNK, HALF), jnp.float32),
        ],
        in_specs=[
            pl.BlockSpec(memory_space=pl.ANY),
            pl.BlockSpec(memory_space=pltpu.MemorySpace.VMEM),
            pl.BlockSpec(memory_space=pl.ANY),
        ],
        out_specs=[pl.BlockSpec(memory_space=pl.ANY)] * 5,
        scratch_shapes=[
            pltpu.MemorySpace.VMEM((PIECE, HALF), jnp.float32),
            pltpu.MemorySpace.VMEM((PIECE, K), jnp.float32),
            pltpu.SemaphoreType.DMA((2, N_STEP, PIECES)),
            pltpu.SemaphoreType.DMA((2, N_STEP, PIECES)),
            pltpu.SemaphoreType.DMA((2, N_STEP, PIECES)),
            pltpu.SemaphoreType.DMA((2, N_STEP, PIECES)),
            pltpu.SemaphoreType.DMA,
            pltpu.SemaphoreType.DMA,
            pltpu.SemaphoreType.DMA,
        ],
        compiler_params=pltpu.CompilerParams(
            collective_id=0,
            vmem_limit_bytes=100 * 1024 * 1024,
        ),
    )(x, w_mat, p0)
    return out[0]


# device time: 1201471 ns/iter; 1.0084x vs baseline; 1.0084x over previous
import jax
import jax.numpy as jnp
from jax import lax
from jax.experimental import pallas as pl
from jax.experimental.pallas import tpu as pltpu

N_DEV = 4
M = 4096
K = 1024
N = 8192
HALF = N // 2
CHUNK = M // N_DEV
PIECE = 512
PIECES = CHUNK // PIECE
N_STEP = N_DEV - 1


def _gelu(y):
    c = 0.7978845608028654
    return 0.5 * y * (1.0 + jnp.tanh(c * (y + 0.044715 * y * y * y)))


def kernel(x, w_mat):
    me_out = lax.axis_index("i")
    xs = lax.dynamic_slice_in_dim(x, me_out * CHUNK, CHUNK, axis=0)
    p0 = jnp.dot(xs, w_mat, preferred_element_type=jnp.float32)

    def body(x_ref, w_ref, p0_ref, out_ref, sb0, sb1, rb0, rb1, va, vx, vw,
             rs_send, rs_recv, ag_send, ag_recv, sem_a, sem_x, sem_o, sem_w):
        me = lax.axis_index("i")
        right = lax.rem(me + 1, N_DEV)
        left = lax.rem(me + N_DEV - 1, N_DEV)
        ring_nbr = (right, left)
        ring_col0 = (0, HALF)
        ring_rb = (rb0, rb1)
        ring_sb = (sb0, sb1)

        cw = pltpu.make_async_copy(w_ref, vw, sem_w)
        cw.start()

        barrier = pltpu.get_barrier_semaphore()
        for nbr in (left, right):
            pl.semaphore_signal(barrier, inc=1, device_id=(nbr,),
                                device_id_type=pl.DeviceIdType.MESH)
        pl.semaphore_wait(barrier, 2)

        def accum_piece(src_piece, row0, col0, dest_piece, apply_gelu):
            ca = pltpu.make_async_copy(src_piece, va, sem_a)
            cx = pltpu.make_async_copy(
                x_ref.at[pl.ds(row0, PIECE), :], vx, sem_x)
            ca.start()
            cx.start()
            cx.wait()
            pp = jnp.dot(vx[...], vw[:, pl.ds(col0, HALF)],
                         preferred_element_type=jnp.float32)
            ca.wait()
            s = va[...] + pp
            va[...] = _gelu(s) if apply_gelu else s
            co = pltpu.make_async_copy(va, dest_piece, sem_o)
            co.start()
            co.wait()

        descs = {}
        ag_descs = {}

        def start_send(r, s, j, src_piece):
            d = pltpu.make_async_remote_copy(
                src_ref=src_piece,
                dst_ref=ring_rb[r].at[s, pl.ds(j * PIECE, PIECE), :],
                send_sem=rs_send.at[r, s, j],
                recv_sem=rs_recv.at[r, s, j],
                device_id=(ring_nbr[r],),
                device_id_type=pl.DeviceIdType.MESH,
            )
            d.start()
            descs[(r, s, j)] = d

        def start_ag(r, t, j):
            if r == 0:
                g = lax.rem(me + 1 - t + N_DEV, N_DEV)
            else:
                g = lax.rem(me - 1 + t + N_DEV, N_DEV)
            piece = out_ref.at[pl.ds(g * CHUNK + j * PIECE, PIECE),
                               pl.ds(ring_col0[r], HALF)]
            d = pltpu.make_async_remote_copy(
                src_ref=piece, dst_ref=piece,
                send_sem=ag_send.at[r, t, j],
                recv_sem=ag_recv.at[r, t, j],
                device_id=(ring_nbr[r],),
                device_id_type=pl.DeviceIdType.MESH,
            )
            d.start()
            ag_descs[(r, t, j)] = d

        for j in range(PIECES):
            for r in (0, 1):
                start_send(r, 0, j, p0_ref.at[pl.ds(j * PIECE, PIECE),
                                              pl.ds(ring_col0[r], HALF)])

        cw.wait()

        for s in range(N_STEP):
            last = s == N_STEP - 1
            rc = (lax.rem(me - s - 1 + N_DEV, N_DEV),
                  lax.rem(me + s + 1, N_DEV))
            for j in range(PIECES):
                for r in (0, 1):
                    descs[(r, s, j)].wait_recv()
                    src = ring_rb[r].at[s, pl.ds(j * PIECE, PIECE), :]
                    row0 = rc[r] * CHUNK + j * PIECE
                    if not last:
                        dest = ring_sb[r].at[s, pl.ds(j * PIECE, PIECE), :]
                        accum_piece(src, row0, ring_col0[r], dest, False)
                        start_send(r, s + 1, j, dest)
                    else:
                        dest = out_ref.at[pl.ds(row0, PIECE),
                                          pl.ds(ring_col0[r], HALF)]
                        accum_piece(src, row0, ring_col0[r], dest, True)
                        start_ag(r, 0, j)
        for d in descs.values():
            d.wait_send()

        for t in range(1, N_STEP):
            for j in range(PIECES):
                for r in (0, 1):
                    ag_descs[(r, t - 1, j)].wait_recv()
                    start_ag(r, t, j)
        for j in range(PIECES):
            for r in (0, 1):
                ag_descs[(r, N_STEP - 1, j)].wait_recv()
        for d in ag_descs.values():
            d.wait_send()

        def _exit(second_barrier):
            for nbr in (left, right):
                pl.semaphore_signal(second_barrier, inc=1, device_id=(nbr,),
                                    device_id_type=pl.DeviceIdType.MESH)
            pl.semaphore_wait(second_barrier, 2)
        pl.run_scoped(_exit, second_barrier=pltpu.SemaphoreType.REGULAR)

    out = pl.pallas_call(
        body,
        out_shape=[
            jax.ShapeDtypeStruct((M, N), jnp.float32),
            jax.ShapeDtypeStruct((N_STEP - 1, CHUNK, HALF), jnp.float32),
            jax.ShapeDtypeStruct((N_STEP - 1, CHUNK, HALF), jnp.float32),
            jax.ShapeDtypeStruct((N_STEP, CHUNK, HALF), jnp.float32),
            jax.ShapeDtypeStruct((N_STEP, CHUNK, HALF), jnp.float32),
        ],
        in_specs=[
            pl.BlockSpec(memory_space=pl.ANY),
            pl.BlockSpec(memory_space=pl.ANY),
            pl.BlockSpec(memory_space=pl.ANY),
        ],
        out_specs=[pl.BlockSpec(memory_space=pl.ANY)] * 5,
        scratch_shapes=[
            pltpu.MemorySpace.VMEM((PIECE, HALF), jnp.float32),
            pltpu.MemorySpace.VMEM((PIECE, K), jnp.float32),
            pltpu.MemorySpace.VMEM((K, N), jnp.float32),
            pltpu.SemaphoreType.DMA((2, N_STEP, PIECES)),
            pltpu.SemaphoreType.DMA((2, N_STEP, PIECES)),
            pltpu.SemaphoreType.DMA((2, N_STEP, PIECES)),
            pltpu.SemaphoreType.DMA((2, N_STEP, PIECES)),
            pltpu.SemaphoreType.DMA,
            pltpu.SemaphoreType.DMA,
            pltpu.SemaphoreType.DMA,
            pltpu.SemaphoreType.DMA,
        ],
        compiler_params=pltpu.CompilerParams(
            collective_id=0,
            vmem_limit_bytes=100 * 1024 * 1024,
        ),
    )(x, w_mat, p0)
    return out[0]
